# baseline (speedup 1.0000x reference)
import functools

import jax
import jax.numpy as jnp
from jax import lax
from jax.experimental import pallas as pl
from jax.experimental.pallas import tpu as pltpu
from jax.experimental.pallas import tpu_sc as plsc

_OUT_D, _OUT_H, _OUT_W = 4, 7, 7
_SCALE = 0.25
_SR = 2
_CELLS2 = _OUT_H * _OUT_W
_TAPS2 = (_SR * 2) * (_SR * 2)
_GROUP = 7
_GROUPS = _CELLS2 // _GROUP
_GROWS = _GROUP * _TAPS2
_NZMAX = 16
_LANES = 16


def _axis_tab(start, extent, nbins, size):
    binsz = extent / nbins
    p = jnp.arange(nbins, dtype=jnp.float32)
    i = (jnp.arange(_SR, dtype=jnp.float32) + 0.5) / _SR
    off = (p[:, None] + i[None, :]).reshape(-1)
    c = start[:, None] + off[None, :] * binsz[:, None]
    valid = (c >= -1.0) & (c <= float(size))
    c = jnp.clip(c, 0.0, float(size - 1))
    lo = jnp.floor(c).astype(jnp.int32)
    lo = jnp.minimum(lo, size - 1)
    hi = jnp.minimum(lo + 1, size - 1)
    frac = c - lo.astype(jnp.float32)
    v = valid.astype(jnp.float32)
    return lo, hi, (1.0 - frac) * v, frac * v


def _build_tabs(rois, N, D, H, W):
    R = rois.shape[0]
    b = rois[:, 0].astype(jnp.int32)
    z1 = rois[:, 1] * _SCALE
    y1 = rois[:, 2] * _SCALE
    x1 = rois[:, 3] * _SCALE
    z2 = rois[:, 4] * _SCALE
    y2 = rois[:, 5] * _SCALE
    x2 = rois[:, 6] * _SCALE
    rd = jnp.maximum(z2 - z1, 1.0)
    rh = jnp.maximum(y2 - y1, 1.0)
    rw = jnp.maximum(x2 - x1, 1.0)
    zlo, zhi, wz0, wz1 = _axis_tab(z1, rd, _OUT_D, D)
    ylo, yhi, wy0, wy1 = _axis_tab(y1, rh, _OUT_H, H)
    xlo, xhi, wx0, wx1 = _axis_tab(x1, rw, _OUT_W, W)

    ysel = jnp.stack([ylo, yhi], -1)
    wyse = jnp.stack([wy0, wy1], -1)
    xsel = jnp.stack([xlo, xhi], -1)
    wxse = jnp.stack([wx0, wx1], -1)
    idxy = (ysel[:, :, None, :, None] * W
            + xsel[:, None, :, None, :])
    wxy = (wyse[:, :, None, :, None]
           * wxse[:, None, :, None, :]) * (1.0 / (_SR ** 3))

    def regroup(a):
        a = a.reshape(R, _OUT_H, _SR, _OUT_W, _SR, 2, 2)
        a = a.transpose(0, 1, 3, 2, 4, 5, 6)
        return a.reshape(R, _CELLS2 * _TAPS2)

    zsel = jnp.stack([zlo, zhi], -1)
    wzse = jnp.stack([wz0, wz1], -1)
    oh = (zsel[..., None] == jnp.arange(D, dtype=jnp.int32))
    wz8 = (oh * wzse[..., None]).sum(2)
    wzpd = wz8.reshape(R, _OUT_D, _SR, D).sum(2)
    wz = jnp.swapaxes(wzpd, 1, 2)
    wz = jnp.pad(wz, ((0, 0), (0, 0), (0, _LANES - _OUT_D)))
    wz = wz.reshape(R, D * _LANES)

    zmin = zsel.min(axis=(1, 2))
    zmax = zsel.max(axis=(1, 2))
    nz = zmax - zmin + 1
    scal = jnp.stack(
        [nz] + [jnp.zeros_like(nz)] * 15, axis=1)

    wzd = wz.reshape(R, D, _LANES)
    wzd = jnp.pad(wzd, ((0, 0), (0, _NZMAX - 1), (0, 0)))
    zr = (zmin[:, None] + jnp.arange(_NZMAX))[..., None]
    wzrel = jnp.take_along_axis(wzd, zr, axis=1).reshape(R, _NZMAX * _LANES)

    idxy_g = regroup(idxy)
    zabs = jnp.minimum(zmin[:, None] + jnp.arange(_NZMAX), D - 1)
    base = b[:, None] * (D * H * W) + zabs * (H * W)
    idx_all = (base[:, :, None] + idxy_g[:, None, :]).reshape(
        R, _NZMAX * _CELLS2 * _TAPS2)

    return idx_all, regroup(wxy), wzrel, scal


def _roi_align_sc(table, idx, wxy, wz, scal, R, C):
    info = plsc.get_sparse_core_info()
    NW = info.num_cores * info.num_subcores
    NC = info.num_cores
    rpw = R // NW
    mesh = plsc.VectorSubcoreMesh(core_axis_name="c", subcore_axis_name="s")
    cbn = C // _LANES
    cells = _OUT_D * _CELLS2

    @functools.partial(
        pl.kernel,
        mesh=mesh,
        compiler_params=pltpu.CompilerParams(use_tc_tiling_on_sc=False),
        out_type=jax.ShapeDtypeStruct((R, cells, C), jnp.float32),
        scratch_types=[
            pltpu.VMEM((_NZMAX * _CELLS2 * _TAPS2,), jnp.int32),
            pltpu.VMEM((_CELLS2 * _TAPS2,), jnp.float32),
            pltpu.VMEM((_NZMAX * _LANES,), jnp.float32),
            pltpu.VMEM((_LANES,), jnp.int32),
            pltpu.VMEM((_GROWS, C // 2), jnp.int32),
            pltpu.VMEM((_GROWS, C // 2), jnp.int32),
            pltpu.VMEM((cells, C), jnp.float32),
            pltpu.SemaphoreType.DMA,
            pltpu.SemaphoreType.DMA,
        ],
    )
    def k(table_hbm, idx_hbm, wxy_hbm, wz_hbm, scal_hbm, out_hbm,
          idx_all, wxy_v, wz_v, scal_v, buf0, buf1, out_v,
          sem0, sem1):
        wid = lax.axis_index("s") * NC + lax.axis_index("c")

        def issue(step, buf, sem):
            pltpu.async_copy(
                table_hbm.at[idx_all.at[pl.ds(step * _GROWS, _GROWS)]],
                buf, sem)

        def drain(step, buf, sem):
            pltpu.make_async_copy(
                table_hbm.at[idx_all.at[pl.ds(step * _GROWS, _GROWS)]],
                buf, sem).wait()

        def compute(step, buf):
            z = step // _GROUPS
            g = step - z * _GROUPS
            wzvec = wz_v[pl.ds(z * _LANES, _LANES)]
            wzs = [wzvec[pd] for pd in range(_OUT_D)]
            for cc in range(_GROUP):
                accs = [jnp.zeros((_LANES,), jnp.float32) for _ in range(cbn)]
                w16 = wxy_v[pl.ds((g * _GROUP + cc) * _TAPS2, _TAPS2)]
                himask = jnp.full((_LANES,), -65536, jnp.int32)
                for j in range(_TAPS2):
                    w = w16[j]
                    r = cc * _TAPS2 + j
                    for k in range(cbn // 2):
                        ab = buf[r, pl.ds(k * _LANES, _LANES)]
                        a = lax.bitcast_convert_type(
                            ab << 16, jnp.float32)
                        b = lax.bitcast_convert_type(
                            ab & himask, jnp.float32)
                        accs[2 * k] += w * a
                        accs[2 * k + 1] += w * b
                cell = g * _GROUP + cc
                for pd in range(_OUT_D):
                    orow = pd * _CELLS2 + cell
                    for cb in range(cbn):
                        sl = pl.ds(cb * _LANES, _LANES)
                        out_v[orow, sl] = out_v[orow, sl] + wzs[pd] * accs[cb]

        def roi_body(rr, _):
            roi = wid * rpw + rr
            pltpu.sync_copy(idx_hbm.at[roi], idx_all)
            pltpu.sync_copy(wxy_hbm.at[roi], wxy_v)
            pltpu.sync_copy(wz_hbm.at[roi], wz_v)
            pltpu.sync_copy(scal_hbm.at[roi], scal_v)

            zeros = jnp.zeros((_LANES,), jnp.float32)

            def zero_body(i, _):
                for cb in range(cbn):
                    out_v[i, pl.ds(cb * _LANES, _LANES)] = zeros
                return 0

            lax.fori_loop(0, cells, zero_body, 0)

            scal16 = scal_v[pl.ds(0, _LANES)]
            nz = scal16[0]
            total = nz * _GROUPS
            issue(0, buf0, sem0)
            issue(1, buf1, sem1)

            def step_body(i2, _):
                s0 = i2 * 2
                drain(s0, buf0, sem0)
                compute(s0, buf0)

                @pl.when(s0 + 2 < total)
                def _():
                    issue(s0 + 2, buf0, sem0)

                @pl.when(s0 + 1 < total)
                def _():
                    drain(s0 + 1, buf1, sem1)
                    compute(s0 + 1, buf1)

                    @pl.when(s0 + 3 < total)
                    def _():
                        issue(s0 + 3, buf1, sem1)

                return 0

            lax.fori_loop(0, (total + 1) // 2, step_body, 0)
            pltpu.sync_copy(out_v, out_hbm.at[roi])
            return 0

        lax.fori_loop(0, rpw, roi_body, 0)

    return k(table, idx, wxy, wz, scal)


def kernel(x, rois):
    N, C, D, H, W = x.shape
    R = rois.shape[0]
    perm32 = jnp.stack(
        [jnp.arange(16), 16 + jnp.arange(16)], axis=1).reshape(32)
    perm = (jnp.arange(0, C, 32)[:, None] + perm32[None, :]).reshape(C)
    table = jnp.transpose(x, (0, 2, 3, 4, 1)).reshape(N * D * H * W, C)
    table = table[:, perm].astype(jnp.bfloat16)
    table = jax.lax.bitcast_convert_type(
        table.reshape(N * D * H * W, C // 2, 2), jnp.int32)
    idxy, wxy, wz, scal = _build_tabs(rois, N, D, H, W)
    out = _roi_align_sc(table, idxy, wxy, wz, scal, R, C)
    return out.reshape(R, _OUT_D, _OUT_H, _OUT_W, C).transpose(0, 4, 1, 2, 3)


# speedup vs baseline: 1.0063x; 1.0063x over previous
import functools

import jax
import jax.numpy as jnp
from jax import lax
from jax.experimental import pallas as pl
from jax.experimental.pallas import tpu as pltpu
from jax.experimental.pallas import tpu_sc as plsc

_OUT_D, _OUT_H, _OUT_W = 4, 7, 7
_SCALE = 0.25
_SR = 2
_CELLS2 = _OUT_H * _OUT_W
_TAPS2 = (_SR * 2) * (_SR * 2)
_GROUP = 7
_GROUPS = _CELLS2 // _GROUP
_GROWS = _GROUP * _TAPS2
_NZMAX = 16
_LANES = 16


def _axis_tab(start, extent, nbins, size):
    binsz = extent / nbins
    p = jnp.arange(nbins, dtype=jnp.float32)
    i = (jnp.arange(_SR, dtype=jnp.float32) + 0.5) / _SR
    off = (p[:, None] + i[None, :]).reshape(-1)
    c = start[:, None] + off[None, :] * binsz[:, None]
    valid = (c >= -1.0) & (c <= float(size))
    c = jnp.clip(c, 0.0, float(size - 1))
    lo = jnp.floor(c).astype(jnp.int32)
    lo = jnp.minimum(lo, size - 1)
    hi = jnp.minimum(lo + 1, size - 1)
    frac = c - lo.astype(jnp.float32)
    v = valid.astype(jnp.float32)
    return lo, hi, (1.0 - frac) * v, frac * v


def _build_tabs(rois, N, D, H, W):
    R = rois.shape[0]
    b = rois[:, 0].astype(jnp.int32)
    z1 = rois[:, 1] * _SCALE
    y1 = rois[:, 2] * _SCALE
    x1 = rois[:, 3] * _SCALE
    z2 = rois[:, 4] * _SCALE
    y2 = rois[:, 5] * _SCALE
    x2 = rois[:, 6] * _SCALE
    rd = jnp.maximum(z2 - z1, 1.0)
    rh = jnp.maximum(y2 - y1, 1.0)
    rw = jnp.maximum(x2 - x1, 1.0)
    zlo, zhi, wz0, wz1 = _axis_tab(z1, rd, _OUT_D, D)
    ylo, yhi, wy0, wy1 = _axis_tab(y1, rh, _OUT_H, H)
    xlo, xhi, wx0, wx1 = _axis_tab(x1, rw, _OUT_W, W)

    ysel = jnp.stack([ylo, yhi], -1)
    wyse = jnp.stack([wy0, wy1], -1)
    xsel = jnp.stack([xlo, xhi], -1)
    wxse = jnp.stack([wx0, wx1], -1)
    idxy = (ysel[:, :, None, :, None] * W
            + xsel[:, None, :, None, :])
    wxy = (wyse[:, :, None, :, None]
           * wxse[:, None, :, None, :]) * (1.0 / (_SR ** 3))

    def regroup(a):
        a = a.reshape(R, _OUT_H, _SR, _OUT_W, _SR, 2, 2)
        a = a.transpose(0, 1, 3, 2, 4, 5, 6)
        return a.reshape(R, _CELLS2 * _TAPS2)

    zsel = jnp.stack([zlo, zhi], -1)
    wzse = jnp.stack([wz0, wz1], -1)
    oh = (zsel[..., None] == jnp.arange(D, dtype=jnp.int32))
    wz8 = (oh * wzse[..., None]).sum(2)
    wzpd = wz8.reshape(R, _OUT_D, _SR, D).sum(2)
    wz = jnp.swapaxes(wzpd, 1, 2)
    wz = jnp.pad(wz, ((0, 0), (0, 0), (0, _LANES - _OUT_D)))
    wz = wz.reshape(R, D * _LANES)

    zmin = zsel.min(axis=(1, 2))
    zmax = zsel.max(axis=(1, 2))
    nz = zmax - zmin + 1
    scal = jnp.stack(
        [nz] + [jnp.zeros_like(nz)] * 15, axis=1)

    wzd = wz.reshape(R, D, _LANES)
    wzd = jnp.pad(wzd, ((0, 0), (0, _NZMAX - 1), (0, 0)))
    zr = (zmin[:, None] + jnp.arange(_NZMAX))[..., None]
    wzrel = jnp.take_along_axis(wzd, zr, axis=1).reshape(R, _NZMAX * _LANES)

    idxy_g = regroup(idxy)
    zabs = jnp.minimum(zmin[:, None] + jnp.arange(_NZMAX), D - 1)
    base = b[:, None] * (D * H * W) + zabs * (H * W)
    idx_all = (base[:, :, None] + idxy_g[:, None, :]).reshape(
        R, _NZMAX * _CELLS2 * _TAPS2)

    return idx_all, regroup(wxy), wzrel, scal


def _roi_align_sc(table, idx, wxy, wz, scal, R, C):
    info = plsc.get_sparse_core_info()
    NW = info.num_cores * info.num_subcores
    NC = info.num_cores
    rpw = R // NW
    mesh = plsc.VectorSubcoreMesh(core_axis_name="c", subcore_axis_name="s")
    cbn = C // _LANES
    cells = _OUT_D * _CELLS2

    @functools.partial(
        pl.kernel,
        mesh=mesh,
        compiler_params=pltpu.CompilerParams(use_tc_tiling_on_sc=False),
        out_type=jax.ShapeDtypeStruct((R, cells, C), jnp.float32),
        scratch_types=[
            pltpu.VMEM((_NZMAX * _CELLS2 * _TAPS2,), jnp.int32),
            pltpu.VMEM((_CELLS2 * _TAPS2,), jnp.float32),
            pltpu.VMEM((_NZMAX * _LANES,), jnp.float32),
            pltpu.VMEM((_LANES,), jnp.int32),
            pltpu.VMEM((_GROWS, C // 2), jnp.int32),
            pltpu.VMEM((_GROWS, C // 2), jnp.int32),
            pltpu.VMEM((cells, C), jnp.float32),
            pltpu.SemaphoreType.DMA,
            pltpu.SemaphoreType.DMA,
        ],
    )
    def k(table_hbm, idx_hbm, wxy_hbm, wz_hbm, scal_hbm, out_hbm,
          idx_all, wxy_v, wz_v, scal_v, buf0, buf1, out_v,
          sem0, sem1):
        wid = lax.axis_index("s") * NC + lax.axis_index("c")

        def issue(step, buf, sem):
            pltpu.async_copy(
                table_hbm.at[idx_all.at[pl.ds(step * _GROWS, _GROWS)]],
                buf, sem)

        def drain(step, buf, sem):
            pltpu.make_async_copy(
                table_hbm.at[idx_all.at[pl.ds(step * _GROWS, _GROWS)]],
                buf, sem).wait()

        def compute(step, buf):
            z = step // _GROUPS
            g = step - z * _GROUPS
            wzvec = wz_v[pl.ds(z * _LANES, _LANES)]
            wzs = [wzvec[pd] for pd in range(_OUT_D)]
            for cc in range(_GROUP):
                accs = [jnp.zeros((_LANES,), jnp.float32) for _ in range(cbn)]
                w16 = wxy_v[pl.ds((g * _GROUP + cc) * _TAPS2, _TAPS2)]
                himask = jnp.full((_LANES,), -65536, jnp.int32)
                for j in range(_TAPS2):
                    w = w16[j]
                    r = cc * _TAPS2 + j
                    for k in range(cbn // 2):
                        ab = buf[r, pl.ds(k * _LANES, _LANES)]
                        a = lax.bitcast_convert_type(
                            ab << 16, jnp.float32)
                        b = lax.bitcast_convert_type(
                            ab & himask, jnp.float32)
                        accs[2 * k] += w * a
                        accs[2 * k + 1] += w * b
                cell = g * _GROUP + cc
                for pd in range(_OUT_D):
                    orow = pd * _CELLS2 + cell
                    for cb in range(cbn):
                        sl = pl.ds(cb * _LANES, _LANES)
                        out_v[orow, sl] = out_v[orow, sl] + wzs[pd] * accs[cb]

        def roi_body(rr, _):
            roi = wid * rpw + rr
            pltpu.sync_copy(idx_hbm.at[roi], idx_all)
            pltpu.sync_copy(wxy_hbm.at[roi], wxy_v)
            pltpu.sync_copy(wz_hbm.at[roi], wz_v)
            pltpu.sync_copy(scal_hbm.at[roi], scal_v)

            zeros = jnp.zeros((_LANES,), jnp.float32)

            def zero_body(i, _):
                for cb in range(cbn):
                    out_v[i, pl.ds(cb * _LANES, _LANES)] = zeros
                return 0

            lax.fori_loop(0, cells, zero_body, 0)

            scal16 = scal_v[pl.ds(0, _LANES)]
            nz = scal16[0]
            total = nz * _GROUPS
            issue(0, buf0, sem0)
            issue(1, buf1, sem1)

            def step_body(i2, _):
                s0 = i2 * 2
                drain(s0, buf0, sem0)
                compute(s0, buf0)

                @pl.when(s0 + 2 < total)
                def _():
                    issue(s0 + 2, buf0, sem0)

                @pl.when(s0 + 1 < total)
                def _():
                    drain(s0 + 1, buf1, sem1)
                    compute(s0 + 1, buf1)

                    @pl.when(s0 + 3 < total)
                    def _():
                        issue(s0 + 3, buf1, sem1)

                return 0

            lax.fori_loop(0, (total + 1) // 2, step_body, 0)
            pltpu.sync_copy(out_v, out_hbm.at[roi])
            return 0

        lax.fori_loop(0, rpw, roi_body, 0)

    return k(table, idx, wxy, wz, scal)


def kernel(x, rois):
    N, C, D, H, W = x.shape
    R = rois.shape[0]
    table = jnp.transpose(x, (0, 2, 3, 4, 1)).reshape(N * D * H * W, C)
    table = jax.lax.bitcast_convert_type(
        table.astype(jnp.bfloat16).reshape(N * D * H * W, C // 2, 2),
        jnp.int32)
    c = jnp.arange(C)
    srcpos = 32 * (c // 32) + 16 * (c % 2) + (c % 32) // 2
    idxy, wxy, wz, scal = _build_tabs(rois, N, D, H, W)
    out = _roi_align_sc(table, idxy, wxy, wz, scal, R, C)
    out = out[:, :, srcpos]
    return out.reshape(R, _OUT_D, _OUT_H, _OUT_W, C).transpose(0, 4, 1, 2, 3)


# speedup vs baseline: 2.1077x; 2.0945x over previous
import functools

import jax
import jax.numpy as jnp
from jax import lax
from jax.experimental import pallas as pl
from jax.experimental.pallas import tpu as pltpu
from jax.experimental.pallas import tpu_sc as plsc

_OUT_D, _OUT_H, _OUT_W = 4, 7, 7
_SCALE = 0.25
_SR = 2
_CELLS2 = _OUT_H * _OUT_W
_TAPS2 = (_SR * 2) * (_SR * 2)
_GROUP = 7
_GROUPS = _CELLS2 // _GROUP
_GROWS = _GROUP * _TAPS2
_NZMAX = 16
_LANES = 16


def _axis_tab(start, extent, nbins, size):
    binsz = extent / nbins
    p = jnp.arange(nbins, dtype=jnp.float32)
    i = (jnp.arange(_SR, dtype=jnp.float32) + 0.5) / _SR
    off = (p[:, None] + i[None, :]).reshape(-1)
    c = start[:, None] + off[None, :] * binsz[:, None]
    valid = (c >= -1.0) & (c <= float(size))
    c = jnp.clip(c, 0.0, float(size - 1))
    lo = jnp.floor(c).astype(jnp.int32)
    lo = jnp.minimum(lo, size - 1)
    hi = jnp.minimum(lo + 1, size - 1)
    frac = c - lo.astype(jnp.float32)
    v = valid.astype(jnp.float32)
    return lo, hi, (1.0 - frac) * v, frac * v


def _build_tabs(rois, N, D, H, W):
    R = rois.shape[0]
    b = rois[:, 0].astype(jnp.int32)
    z1 = rois[:, 1] * _SCALE
    y1 = rois[:, 2] * _SCALE
    x1 = rois[:, 3] * _SCALE
    z2 = rois[:, 4] * _SCALE
    y2 = rois[:, 5] * _SCALE
    x2 = rois[:, 6] * _SCALE
    rd = jnp.maximum(z2 - z1, 1.0)
    rh = jnp.maximum(y2 - y1, 1.0)
    rw = jnp.maximum(x2 - x1, 1.0)
    zlo, zhi, wz0, wz1 = _axis_tab(z1, rd, _OUT_D, D)
    ylo, yhi, wy0, wy1 = _axis_tab(y1, rh, _OUT_H, H)
    xlo, xhi, wx0, wx1 = _axis_tab(x1, rw, _OUT_W, W)

    ysel = jnp.stack([ylo, yhi], -1)
    wyse = jnp.stack([wy0, wy1], -1)
    xsel = jnp.stack([xlo, xhi], -1)
    wxse = jnp.stack([wx0, wx1], -1)
    idxy = (ysel[:, :, None, :, None] * W
            + xsel[:, None, :, None, :])
    wxy = (wyse[:, :, None, :, None]
           * wxse[:, None, :, None, :]) * (1.0 / (_SR ** 3))

    def regroup(a):
        a = a.reshape(R, _OUT_H, _SR, _OUT_W, _SR, 2, 2)
        a = a.transpose(0, 1, 3, 2, 4, 5, 6)
        return a.reshape(R, _CELLS2 * _TAPS2)

    zsel = jnp.stack([zlo, zhi], -1)
    wzse = jnp.stack([wz0, wz1], -1)
    oh = (zsel[..., None] == jnp.arange(D, dtype=jnp.int32))
    wz8 = (oh * wzse[..., None]).sum(2)
    wzpd = wz8.reshape(R, _OUT_D, _SR, D).sum(2)
    wz = jnp.swapaxes(wzpd, 1, 2)
    wz = jnp.pad(wz, ((0, 0), (0, 0), (0, _LANES - _OUT_D)))
    wz = wz.reshape(R, D * _LANES)

    zmin = zsel.min(axis=(1, 2))
    zmax = zsel.max(axis=(1, 2))
    nz = zmax - zmin + 1
    scal = jnp.stack(
        [nz] + [jnp.zeros_like(nz)] * 15, axis=1)

    wzd = wz.reshape(R, D, _LANES)
    wzd = jnp.pad(wzd, ((0, 0), (0, _NZMAX - 1), (0, 0)))
    zr = (zmin[:, None] + jnp.arange(_NZMAX))[..., None]
    wzrel = jnp.take_along_axis(wzd, zr, axis=1).reshape(R, _NZMAX * _LANES)

    idxy_g = regroup(idxy)
    zabs = jnp.minimum(zmin[:, None] + jnp.arange(_NZMAX), D - 1)
    base = b[:, None] * (D * H * W) + zabs * (H * W)
    idx_all = (base[:, :, None] + idxy_g[:, None, :]).reshape(
        R, _NZMAX * _CELLS2 * _TAPS2)

    return idx_all, regroup(wxy), wzrel, scal


def _roi_align_sc(table, idx, wxy, wz, scal, R, C):
    info = plsc.get_sparse_core_info()
    NW = info.num_cores * info.num_subcores
    NC = info.num_cores
    rpw = R // NW
    mesh = plsc.VectorSubcoreMesh(core_axis_name="c", subcore_axis_name="s")
    cbn = C // _LANES
    cells = _OUT_D * _CELLS2

    @functools.partial(
        pl.kernel,
        mesh=mesh,
        out_type=jax.ShapeDtypeStruct((R, cells, C), jnp.float32),
        scratch_types=[
            pltpu.VMEM((_NZMAX * _CELLS2 * _TAPS2,), jnp.int32),
            pltpu.VMEM((_CELLS2 * _TAPS2,), jnp.float32),
            pltpu.VMEM((_NZMAX * _LANES,), jnp.float32),
            pltpu.VMEM((_LANES,), jnp.int32),
            pltpu.VMEM((_GROWS, C), jnp.float32),
            pltpu.VMEM((_GROWS, C), jnp.float32),
            pltpu.VMEM((_GROWS, C), jnp.float32),
            pltpu.VMEM((_GROWS, C), jnp.float32),
            pltpu.VMEM((cells, C), jnp.float32),
            pltpu.SemaphoreType.DMA,
            pltpu.SemaphoreType.DMA,
            pltpu.SemaphoreType.DMA,
            pltpu.SemaphoreType.DMA,
        ],
    )
    def k(table_hbm, idx_hbm, wxy_hbm, wz_hbm, scal_hbm, out_hbm,
          idx_all, wxy_v, wz_v, scal_v, buf0, buf1, buf2, buf3, out_v,
          sem0, sem1, sem2, sem3):
        wid = lax.axis_index("s") * NC + lax.axis_index("c")
        bufs = [buf0, buf1, buf2, buf3]
        sems = [sem0, sem1, sem2, sem3]

        def issue(step, buf, sem):
            pltpu.async_copy(
                table_hbm.at[idx_all.at[pl.ds(step * _GROWS, _GROWS)]],
                buf, sem)

        def drain(step, buf, sem):
            pltpu.make_async_copy(
                table_hbm.at[idx_all.at[pl.ds(step * _GROWS, _GROWS)]],
                buf, sem).wait()

        def compute(step, buf):
            z = step // _GROUPS
            g = step - z * _GROUPS
            wzvec = wz_v[pl.ds(z * _LANES, _LANES)]
            wzs = [wzvec[pd] for pd in range(_OUT_D)]
            for cc in range(_GROUP):
                accs = [jnp.zeros((_LANES,), jnp.float32) for _ in range(cbn)]
                w16 = wxy_v[pl.ds((g * _GROUP + cc) * _TAPS2, _TAPS2)]
                for j in range(_TAPS2):
                    w = w16[j]
                    r = cc * _TAPS2 + j
                    for cb in range(cbn):
                        accs[cb] += w * buf[r, pl.ds(cb * _LANES, _LANES)]
                cell = g * _GROUP + cc
                for pd in range(_OUT_D):
                    orow = pd * _CELLS2 + cell
                    for cb in range(cbn):
                        sl = pl.ds(cb * _LANES, _LANES)
                        out_v[orow, sl] = out_v[orow, sl] + wzs[pd] * accs[cb]

        def roi_body(rr, _):
            roi = wid * rpw + rr
            pltpu.sync_copy(idx_hbm.at[roi], idx_all)
            pltpu.sync_copy(wxy_hbm.at[roi], wxy_v)
            pltpu.sync_copy(wz_hbm.at[roi], wz_v)
            pltpu.sync_copy(scal_hbm.at[roi], scal_v)

            zeros = jnp.zeros((_LANES,), jnp.float32)

            def zero_body(i, _):
                for cb in range(cbn):
                    out_v[i, pl.ds(cb * _LANES, _LANES)] = zeros
                return 0

            lax.fori_loop(0, cells, zero_body, 0)

            scal16 = scal_v[pl.ds(0, _LANES)]
            nz = scal16[0]
            total = nz * _GROUPS
            for u in range(4):
                issue(u, bufs[u], sems[u])

            def step_body(i4, _):
                s0 = i4 * 4
                for u in range(4):
                    s = s0 + u

                    @pl.when(s < total)
                    def _(s=s, u=u):
                        drain(s, bufs[u], sems[u])
                        compute(s, bufs[u])

                        @pl.when(s + 4 < total)
                        def _():
                            issue(s + 4, bufs[u], sems[u])

                return 0

            lax.fori_loop(0, (total + 3) // 4, step_body, 0)
            pltpu.sync_copy(out_v, out_hbm.at[roi])
            return 0

        lax.fori_loop(0, rpw, roi_body, 0)

    return k(table, idx, wxy, wz, scal)


def kernel(x, rois):
    N, C, D, H, W = x.shape
    R = rois.shape[0]
    table = jnp.transpose(x, (0, 2, 3, 4, 1)).reshape(N * D * H * W, C)
    idxy, wxy, wz, scal = _build_tabs(rois, N, D, H, W)
    out = _roi_align_sc(table, idxy, wxy, wz, scal, R, C)
    return out.reshape(R, _OUT_D, _OUT_H, _OUT_W, C).transpose(0, 4, 1, 2, 3)


# speedup vs baseline: 2.3667x; 1.1229x over previous
import functools

import jax
import jax.numpy as jnp
from jax import lax
from jax.experimental import pallas as pl
from jax.experimental.pallas import tpu as pltpu
from jax.experimental.pallas import tpu_sc as plsc

_OUT_D, _OUT_H, _OUT_W = 4, 7, 7
_SCALE = 0.25
_SR = 2
_CELLS2 = _OUT_H * _OUT_W
_TAPS2 = (_SR * 2) * (_SR * 2)
_GROUP = 7
_GROUPS = _CELLS2 // _GROUP
_GROWS = _GROUP * _TAPS2
_NZMAX = 16
_LANES = 16


def _axis_tab(start, extent, nbins, size):
    binsz = extent / nbins
    p = jnp.arange(nbins, dtype=jnp.float32)
    i = (jnp.arange(_SR, dtype=jnp.float32) + 0.5) / _SR
    off = (p[:, None] + i[None, :]).reshape(-1)
    c = start[:, None] + off[None, :] * binsz[:, None]
    valid = (c >= -1.0) & (c <= float(size))
    c = jnp.clip(c, 0.0, float(size - 1))
    lo = jnp.floor(c).astype(jnp.int32)
    lo = jnp.minimum(lo, size - 1)
    hi = jnp.minimum(lo + 1, size - 1)
    frac = c - lo.astype(jnp.float32)
    v = valid.astype(jnp.float32)
    return lo, hi, (1.0 - frac) * v, frac * v


def _build_tabs(rois, N, D, H, W):
    R = rois.shape[0]
    b = rois[:, 0].astype(jnp.int32)
    z1 = rois[:, 1] * _SCALE
    y1 = rois[:, 2] * _SCALE
    x1 = rois[:, 3] * _SCALE
    z2 = rois[:, 4] * _SCALE
    y2 = rois[:, 5] * _SCALE
    x2 = rois[:, 6] * _SCALE
    rd = jnp.maximum(z2 - z1, 1.0)
    rh = jnp.maximum(y2 - y1, 1.0)
    rw = jnp.maximum(x2 - x1, 1.0)
    zlo, zhi, wz0, wz1 = _axis_tab(z1, rd, _OUT_D, D)
    ylo, yhi, wy0, wy1 = _axis_tab(y1, rh, _OUT_H, H)
    xlo, xhi, wx0, wx1 = _axis_tab(x1, rw, _OUT_W, W)

    ysel = jnp.stack([ylo, yhi], -1)
    wyse = jnp.stack([wy0, wy1], -1)
    xsel = jnp.stack([xlo, xhi], -1)
    wxse = jnp.stack([wx0, wx1], -1)
    idxy = (ysel[:, :, None, :, None] * W
            + xsel[:, None, :, None, :])
    wxy = (wyse[:, :, None, :, None]
           * wxse[:, None, :, None, :]) * (1.0 / (_SR ** 3))

    def regroup(a):
        a = a.reshape(R, _OUT_H, _SR, _OUT_W, _SR, 2, 2)
        a = a.transpose(0, 1, 3, 2, 4, 5, 6)
        return a.reshape(R, _CELLS2 * _TAPS2)

    zsel = jnp.stack([zlo, zhi], -1)
    wzse = jnp.stack([wz0, wz1], -1)
    oh = (zsel[..., None] == jnp.arange(D, dtype=jnp.int32))
    wz8 = (oh * wzse[..., None]).sum(2)
    wzpd = wz8.reshape(R, _OUT_D, _SR, D).sum(2)
    wz = jnp.swapaxes(wzpd, 1, 2)
    wz = jnp.pad(wz, ((0, 0), (0, 0), (0, _LANES - _OUT_D)))
    wz = wz.reshape(R, D * _LANES)

    zmin = zsel.min(axis=(1, 2))
    zmax = zsel.max(axis=(1, 2))
    nz = zmax - zmin + 1
    scal = jnp.stack(
        [nz] + [jnp.zeros_like(nz)] * 15, axis=1)

    wzd = wz.reshape(R, D, _LANES)
    wzd = jnp.pad(wzd, ((0, 0), (0, _NZMAX - 1), (0, 0)))
    zr = (zmin[:, None] + jnp.arange(_NZMAX))[..., None]
    wzrel = jnp.take_along_axis(wzd, zr, axis=1).reshape(R, _NZMAX * _LANES)

    idxy_g = regroup(idxy)
    zabs = jnp.minimum(zmin[:, None] + jnp.arange(_NZMAX), D - 1)
    base = b[:, None] * (D * H * W) + zabs * (H * W)
    idx_all = (base[:, :, None] + idxy_g[:, None, :]).reshape(
        R, _NZMAX * _CELLS2 * _TAPS2)

    return idx_all, regroup(wxy), wzrel, scal


def _roi_align_sc(table, idx, wxy, wz, scal, R, C):
    info = plsc.get_sparse_core_info()
    NW = info.num_cores * info.num_subcores
    NC = info.num_cores
    rpw = R // NW
    mesh = plsc.VectorSubcoreMesh(core_axis_name="c", subcore_axis_name="s")
    cbn = C // _LANES
    cells = _OUT_D * _CELLS2

    @functools.partial(
        pl.kernel,
        mesh=mesh,
        out_type=jax.ShapeDtypeStruct((R, cells, C), jnp.float32),
        scratch_types=[
            pltpu.VMEM((_NZMAX * _CELLS2 * _TAPS2,), jnp.int32),
            pltpu.VMEM((_CELLS2 * _TAPS2,), jnp.float32),
            pltpu.VMEM((_NZMAX * _LANES,), jnp.float32),
            pltpu.VMEM((_LANES,), jnp.int32),
            pltpu.VMEM((_GROWS, C), jnp.float32),
            pltpu.VMEM((_GROWS, C), jnp.float32),
            pltpu.VMEM((cells, C), jnp.float32),
            pltpu.SemaphoreType.DMA,
            pltpu.SemaphoreType.DMA,
        ],
    )
    def k(table_hbm, idx_hbm, wxy_hbm, wz_hbm, scal_hbm, out_hbm,
          idx_all, wxy_v, wz_v, scal_v, buf0, buf1, out_v,
          sem0, sem1):
        wid = lax.axis_index("s") * NC + lax.axis_index("c")

        def issue(step, buf, sem):
            pltpu.async_copy(
                table_hbm.at[idx_all.at[pl.ds(step * _GROWS, _GROWS)]],
                buf, sem)

        def drain(step, buf, sem):
            pltpu.make_async_copy(
                table_hbm.at[idx_all.at[pl.ds(step * _GROWS, _GROWS)]],
                buf, sem).wait()

        def compute(step, buf):
            z = step // _GROUPS
            g = step - z * _GROUPS
            wzvec = wz_v[pl.ds(z * _LANES, _LANES)]
            wzs = [wzvec[pd] for pd in range(_OUT_D)]
            for cc in range(_GROUP):
                accs = [jnp.zeros((_LANES,), jnp.float32) for _ in range(cbn)]
                w16 = wxy_v[pl.ds((g * _GROUP + cc) * _TAPS2, _TAPS2)]
                for j in range(_TAPS2):
                    w = w16[j]
                    r = cc * _TAPS2 + j
                    for cb in range(cbn):
                        accs[cb] += w * buf[r, pl.ds(cb * _LANES, _LANES)]
                cell = g * _GROUP + cc
                for pd in range(_OUT_D):
                    orow = pd * _CELLS2 + cell
                    for cb in range(cbn):
                        sl = pl.ds(cb * _LANES, _LANES)
                        out_v[orow, sl] = out_v[orow, sl] + wzs[pd] * accs[cb]

        def roi_body(rr, _):
            roi = wid * rpw + rr
            pltpu.sync_copy(idx_hbm.at[roi], idx_all)
            pltpu.sync_copy(wxy_hbm.at[roi], wxy_v)
            pltpu.sync_copy(wz_hbm.at[roi], wz_v)
            pltpu.sync_copy(scal_hbm.at[roi], scal_v)

            zeros = jnp.zeros((_LANES,), jnp.float32)

            def zero_body(i, _):
                for cb in range(cbn):
                    out_v[i, pl.ds(cb * _LANES, _LANES)] = zeros
                return 0

            lax.fori_loop(0, cells, zero_body, 0)

            scal16 = scal_v[pl.ds(0, _LANES)]
            nz = scal16[0]
            total = nz * _GROUPS
            issue(0, buf0, sem0)
            issue(1, buf1, sem1)

            def step_body(i2, _):
                s0 = i2 * 2
                drain(s0, buf0, sem0)
                compute(s0, buf0)

                @pl.when(s0 + 2 < total)
                def _():
                    issue(s0 + 2, buf0, sem0)

                @pl.when(s0 + 1 < total)
                def _():
                    drain(s0 + 1, buf1, sem1)
                    compute(s0 + 1, buf1)

                    @pl.when(s0 + 3 < total)
                    def _():
                        issue(s0 + 3, buf1, sem1)

                return 0

            lax.fori_loop(0, (total + 1) // 2, step_body, 0)
            pltpu.sync_copy(out_v, out_hbm.at[roi])
            return 0

        lax.fori_loop(0, rpw, roi_body, 0)

    return k(table, idx, wxy, wz, scal)


def kernel(x, rois):
    N, C, D, H, W = x.shape
    R = rois.shape[0]
    table = jnp.transpose(x, (0, 2, 3, 4, 1)).reshape(N * D * H * W, C)
    idxy, wxy, wz, scal = _build_tabs(rois, N, D, H, W)
    out = _roi_align_sc(table, idxy, wxy, wz, scal, R, C)
    return out.reshape(R, _OUT_D, _OUT_H, _OUT_W, C).transpose(0, 4, 1, 2, 3)
